# Initial kernel scaffold; baseline (speedup 1.0000x reference)
import jax
import jax.numpy as jnp
from jax.experimental import pallas as pl


def kernel(x):
    raise NotImplementedError("write your pallas kernel here")


# speedup vs baseline: 3.1022x; 3.1022x over previous
import jax
import jax.numpy as jnp
from jax.experimental import pallas as pl


def _pair_kernel(x_ref, o_ref):
    xb = x_ref[...]
    F = xb.shape[1]
    cols = []
    for i in range(F - 1):
        xi = xb[:, i : i + 1, :]
        prod = xi * xb[:, i + 1 :, :]
        cols.append(jnp.sum(prod, axis=-1))
    o_ref[...] = jnp.concatenate(cols, axis=-1)


def kernel(x):
    N, F, D = x.shape
    P = F * (F - 1) // 2
    B = 128
    return pl.pallas_call(
        _pair_kernel,
        grid=(N // B,),
        in_specs=[pl.BlockSpec((B, F, D), lambda n: (n, 0, 0))],
        out_specs=pl.BlockSpec((B, P), lambda n: (n, 0)),
        out_shape=jax.ShapeDtypeStruct((N, P), x.dtype),
    )(x)


# speedup vs baseline: 7.9723x; 2.5699x over previous
import jax
import jax.numpy as jnp
from jax.experimental import pallas as pl


def _pair_kernel(x_ref, o_ref):
    xb = x_ref[...]
    F = xb.shape[1]
    xt = [jnp.transpose(xb[:, i, :]) for i in range(F)]
    rows = []
    for i in range(F - 1):
        for j in range(i + 1, F):
            rows.append(jnp.sum(xt[i] * xt[j], axis=0, keepdims=True))
    pt = jnp.concatenate(rows, axis=0)
    o_ref[...] = jnp.transpose(pt)


def kernel(x):
    N, F, D = x.shape
    P = F * (F - 1) // 2
    B = 128
    return pl.pallas_call(
        _pair_kernel,
        grid=(N // B,),
        in_specs=[pl.BlockSpec((B, F, D), lambda n: (n, 0, 0))],
        out_specs=pl.BlockSpec((B, P), lambda n: (n, 0)),
        out_shape=jax.ShapeDtypeStruct((N, P), x.dtype),
    )(x)
